# baseline (speedup 1.0000x reference)
import functools

import jax
import jax.numpy as jnp
from jax import lax
from jax.experimental import pallas as pl
from jax.experimental.pallas import tpu as pltpu
from jax.experimental.pallas import tpu_sc as plsc

_N_E = 8192
_E_DIM = 64
_BETA = 0.25
_R = 256
_B = 8192
_NB = _B // _R
_N_ELEMS = 8 * 64 * 32 * 32


def _argmin_body(z_ref, emb_ref, idx_ref, loss_ref, acc_ref):
    i = pl.program_id(0)

    @pl.when(i == 0)
    def _init():
        acc_ref[0, 0] = 0.0

    z = z_ref[...]
    emb = emb_ref[...]
    zn = jnp.sum(z * z, axis=1, keepdims=True)
    en = jnp.sum(emb * emb, axis=1)
    m = lax.dot_general(
        z.astype(jnp.bfloat16), emb.astype(jnp.bfloat16),
        (((1,), (1,)), ((), ())),
        preferred_element_type=jnp.float32,
    )
    d = (zn + en[None, :]) - 2.0 * m
    mn = jnp.min(d, axis=1, keepdims=True)
    ii = lax.broadcasted_iota(jnp.int32, d.shape, 1)
    first = jnp.min(jnp.where(d == mn, ii, _N_E), axis=1)
    idx_ref[0, 0, :] = first
    acc_ref[0, 0] += jnp.sum(mn[:, 0])

    @pl.when(i == pl.num_programs(0) - 1)
    def _fin():
        loss_ref[...] = jnp.full(
            (1, 1), acc_ref[0, 0] * ((1.0 + _BETA) / _N_ELEMS), jnp.float32)


_tc_argmin = pl.pallas_call(
    _argmin_body,
    grid=(_NB,),
    in_specs=[
        pl.BlockSpec((_R, _E_DIM), lambda i: (i, 0)),
        pl.BlockSpec((_N_E, _E_DIM), lambda i: (0, 0)),
    ],
    out_specs=[
        pl.BlockSpec((1, 1, _R), lambda i: (i, 0, 0)),
        pl.BlockSpec((1, 1), lambda i: (0, 0)),
    ],
    out_shape=[
        jax.ShapeDtypeStruct((_NB, 1, _R), jnp.int32),
        jax.ShapeDtypeStruct((1, 1), jnp.float32),
    ],
    scratch_shapes=[pltpu.SMEM((1, 1), jnp.float32)],
)


@functools.lru_cache(maxsize=1)
def _make_sc_gather():
    info = plsc.get_sparse_core_info()
    nw = info.num_cores * info.num_subcores
    b_per_w = _B // nw
    mesh = plsc.VectorSubcoreMesh(core_axis_name="c", subcore_axis_name="s")

    @functools.partial(
        pl.kernel,
        mesh=mesh,
        out_type=jax.ShapeDtypeStruct((_B, _E_DIM), jnp.float32),
        scratch_types=[
            pltpu.VMEM((b_per_w,), jnp.int32),
            pltpu.VMEM((b_per_w, _E_DIM), jnp.float32),
            pltpu.SemaphoreType.DMA,
        ],
        compiler_params=pltpu.CompilerParams(use_tc_tiling_on_sc=False),
    )
    def gather(table_hbm, idx_hbm, out_hbm, idx_v, rows_v, sem):
        wid = lax.axis_index("s") * info.num_cores + lax.axis_index("c")
        base = wid * b_per_w
        pltpu.sync_copy(idx_hbm.at[pl.ds(base, b_per_w)], idx_v)
        pltpu.async_copy(table_hbm.at[idx_v], rows_v, sem).wait()
        pltpu.sync_copy(rows_v, out_hbm.at[pl.ds(base, b_per_w)])

    return gather


def kernel(z, emb_weight):
    z = z.astype(jnp.float32)
    z_p = jnp.transpose(z, (0, 2, 3, 1))
    z_flat = z_p.reshape(-1, _E_DIM)
    idx3, loss11 = _tc_argmin(z_flat, emb_weight)
    min_indices = idx3.reshape(-1)
    z_q = _make_sc_gather()(emb_weight, min_indices).reshape(z_p.shape)
    z_q_st = z_p + lax.stop_gradient(z_q - z_p)
    z_q_out = jnp.transpose(z_q_st, (0, 3, 1, 2))
    return (z_q_out, loss11[0, 0], min_indices)


# speedup vs baseline: 1.1776x; 1.1776x over previous
import functools

import jax
import jax.numpy as jnp
from jax import lax
from jax.experimental import pallas as pl
from jax.experimental.pallas import tpu as pltpu
from jax.experimental.pallas import tpu_sc as plsc

_N_E = 8192
_E_DIM = 64
_BETA = 0.25
_R = 512
_B = 8192
_NB = _B // _R
_N_ELEMS = 8 * 64 * 32 * 32


def _argmin_body(z_ref, emb_ref, idx_ref, loss_ref, acc_ref):
    i = pl.program_id(0)

    @pl.when(i == 0)
    def _init():
        acc_ref[0, 0] = 0.0

    z = z_ref[...]
    emb = emb_ref[...]
    zn = jnp.sum(z * z, axis=1, keepdims=True)
    en = jnp.sum(emb * emb, axis=1)
    m = lax.dot_general(
        z.astype(jnp.bfloat16), emb.astype(jnp.bfloat16),
        (((1,), (1,)), ((), ())),
        preferred_element_type=jnp.float32,
    )
    d = (zn + en[None, :]) - 2.0 * m
    mn = jnp.min(d, axis=1, keepdims=True)
    ii = lax.broadcasted_iota(jnp.int32, d.shape, 1)
    first = jnp.min(jnp.where(d == mn, ii, _N_E), axis=1)
    idx_ref[0, 0, :] = first
    acc_ref[0, 0] += jnp.sum(mn[:, 0])

    @pl.when(i == pl.num_programs(0) - 1)
    def _fin():
        loss_ref[...] = jnp.full(
            (1, 1), acc_ref[0, 0] * ((1.0 + _BETA) / _N_ELEMS), jnp.float32)


_tc_argmin = pl.pallas_call(
    _argmin_body,
    grid=(_NB,),
    in_specs=[
        pl.BlockSpec((_R, _E_DIM), lambda i: (i, 0)),
        pl.BlockSpec((_N_E, _E_DIM), lambda i: (0, 0)),
    ],
    out_specs=[
        pl.BlockSpec((1, 1, _R), lambda i: (i, 0, 0)),
        pl.BlockSpec((1, 1), lambda i: (0, 0)),
    ],
    out_shape=[
        jax.ShapeDtypeStruct((_NB, 1, _R), jnp.int32),
        jax.ShapeDtypeStruct((1, 1), jnp.float32),
    ],
    scratch_shapes=[pltpu.SMEM((1, 1), jnp.float32)],
)


@functools.lru_cache(maxsize=1)
def _make_sc_gather():
    info = plsc.get_sparse_core_info()
    nw = info.num_cores * info.num_subcores
    b_per_w = _B // nw
    mesh = plsc.VectorSubcoreMesh(core_axis_name="c", subcore_axis_name="s")

    @functools.partial(
        pl.kernel,
        mesh=mesh,
        out_type=jax.ShapeDtypeStruct((_B, _E_DIM), jnp.float32),
        scratch_types=[
            pltpu.VMEM((b_per_w,), jnp.int32),
            pltpu.VMEM((b_per_w, _E_DIM), jnp.float32),
            pltpu.SemaphoreType.DMA,
        ],
        compiler_params=pltpu.CompilerParams(use_tc_tiling_on_sc=False),
    )
    def gather(table_hbm, idx_hbm, out_hbm, idx_v, rows_v, sem):
        wid = lax.axis_index("s") * info.num_cores + lax.axis_index("c")
        base = wid * b_per_w
        pltpu.sync_copy(idx_hbm.at[pl.ds(base, b_per_w)], idx_v)
        pltpu.async_copy(table_hbm.at[idx_v], rows_v, sem).wait()
        pltpu.sync_copy(rows_v, out_hbm.at[pl.ds(base, b_per_w)])

    return gather


def kernel(z, emb_weight):
    z = z.astype(jnp.float32)
    z_p = jnp.transpose(z, (0, 2, 3, 1))
    z_flat = z_p.reshape(-1, _E_DIM)
    idx3, loss11 = _tc_argmin(z_flat, emb_weight)
    min_indices = idx3.reshape(-1)
    z_q = _make_sc_gather()(emb_weight, min_indices).reshape(z_p.shape)
    z_q_st = z_p + lax.stop_gradient(z_q - z_p)
    z_q_out = jnp.transpose(z_q_st, (0, 3, 1, 2))
    return (z_q_out, loss11[0, 0], min_indices)


# speedup vs baseline: 1.2455x; 1.0576x over previous
import functools

import jax
import jax.numpy as jnp
from jax import lax
from jax.experimental import pallas as pl
from jax.experimental.pallas import tpu as pltpu
from jax.experimental.pallas import tpu_sc as plsc

_N_E = 8192
_E_DIM = 64
_BETA = 0.25
_R = 1024
_B = 8192
_NB = _B // _R
_N_ELEMS = 8 * 64 * 32 * 32


def _argmin_body(z_ref, emb_ref, idx_ref, loss_ref, acc_ref):
    i = pl.program_id(0)

    @pl.when(i == 0)
    def _init():
        acc_ref[0, 0] = 0.0

    z = z_ref[...]
    emb = emb_ref[...]
    zn = jnp.sum(z * z, axis=1, keepdims=True)
    en = jnp.sum(emb * emb, axis=1)
    m = lax.dot_general(
        z.astype(jnp.bfloat16), emb.astype(jnp.bfloat16),
        (((1,), (1,)), ((), ())),
        preferred_element_type=jnp.float32,
    )
    d = (zn + en[None, :]) - 2.0 * m
    mn = jnp.min(d, axis=1, keepdims=True)
    ii = lax.broadcasted_iota(jnp.int32, d.shape, 1)
    first = jnp.min(jnp.where(d == mn, ii, _N_E), axis=1)
    idx_ref[0, 0, :] = first
    acc_ref[0, 0] += jnp.sum(mn[:, 0])

    @pl.when(i == pl.num_programs(0) - 1)
    def _fin():
        loss_ref[...] = jnp.full(
            (1, 1), acc_ref[0, 0] * ((1.0 + _BETA) / _N_ELEMS), jnp.float32)


_tc_argmin = pl.pallas_call(
    _argmin_body,
    grid=(_NB,),
    in_specs=[
        pl.BlockSpec((_R, _E_DIM), lambda i: (i, 0)),
        pl.BlockSpec((_N_E, _E_DIM), lambda i: (0, 0)),
    ],
    out_specs=[
        pl.BlockSpec((1, 1, _R), lambda i: (i, 0, 0)),
        pl.BlockSpec((1, 1), lambda i: (0, 0)),
    ],
    out_shape=[
        jax.ShapeDtypeStruct((_NB, 1, _R), jnp.int32),
        jax.ShapeDtypeStruct((1, 1), jnp.float32),
    ],
    scratch_shapes=[pltpu.SMEM((1, 1), jnp.float32)],
)


@functools.lru_cache(maxsize=1)
def _make_sc_gather():
    info = plsc.get_sparse_core_info()
    nw = info.num_cores * info.num_subcores
    b_per_w = _B // nw
    mesh = plsc.VectorSubcoreMesh(core_axis_name="c", subcore_axis_name="s")

    @functools.partial(
        pl.kernel,
        mesh=mesh,
        out_type=jax.ShapeDtypeStruct((_B, _E_DIM), jnp.float32),
        scratch_types=[
            pltpu.VMEM((b_per_w,), jnp.int32),
            pltpu.VMEM((b_per_w, _E_DIM), jnp.float32),
            pltpu.SemaphoreType.DMA,
        ],
        compiler_params=pltpu.CompilerParams(use_tc_tiling_on_sc=False),
    )
    def gather(table_hbm, idx_hbm, out_hbm, idx_v, rows_v, sem):
        wid = lax.axis_index("s") * info.num_cores + lax.axis_index("c")
        base = wid * b_per_w
        pltpu.sync_copy(idx_hbm.at[pl.ds(base, b_per_w)], idx_v)
        pltpu.async_copy(table_hbm.at[idx_v], rows_v, sem).wait()
        pltpu.sync_copy(rows_v, out_hbm.at[pl.ds(base, b_per_w)])

    return gather


def kernel(z, emb_weight):
    z = z.astype(jnp.float32)
    z_p = jnp.transpose(z, (0, 2, 3, 1))
    z_flat = z_p.reshape(-1, _E_DIM)
    idx3, loss11 = _tc_argmin(z_flat, emb_weight)
    min_indices = idx3.reshape(-1)
    z_q = _make_sc_gather()(emb_weight, min_indices).reshape(z_p.shape)
    z_q_st = z_p + lax.stop_gradient(z_q - z_p)
    z_q_out = jnp.transpose(z_q_st, (0, 3, 1, 2))
    return (z_q_out, loss11[0, 0], min_indices)
